# baseline (speedup 1.0000x reference)
import functools

import jax
import jax.numpy as jnp
from jax import lax
from jax.experimental import pallas as pl
from jax.experimental.pallas import tpu as pltpu
from jax.experimental.pallas import tpu_sc as plsc

N = 10000
E = 160000
D_NODE = 32
D_EDGE = 16
D_OUT = 32
H = D_OUT * D_NODE

NC = 2
NS = 16
NW = NC * NS
EPW = E // NW
CB = 40
NCH = EPW // CB
N_PAD = 10240
NPS = N_PAD // NS

_SC_MESH = dict(core_axis_name="c", subcore_axis_name="s")


def _sc_gather(node_feats, src):
    node_pad = jnp.pad(node_feats, ((0, 0), (0, 128 - D_NODE)))
    src_r = src.reshape(NW, NCH, CB)

    @functools.partial(
        pl.kernel,
        mesh=plsc.VectorSubcoreMesh(**_SC_MESH),
        out_type=jax.ShapeDtypeStruct((E, 128), jnp.float32),
        scratch_types=[
            pltpu.VMEM((NCH, CB), jnp.int32),
            pltpu.VMEM((CB, 128), jnp.float32),
            pltpu.SemaphoreType.DMA,
        ],
    )
    def k(node_hbm, src_hbm, out_hbm, idx_v, rows_v, sem):
        cid = lax.axis_index("c")
        sid = lax.axis_index("s")
        wid = sid * NC + cid
        base = wid * EPW
        pltpu.sync_copy(src_hbm.at[wid], idx_v)

        def body(j, _):
            pltpu.async_copy(node_hbm.at[idx_v.at[j]], rows_v, sem).wait()
            pltpu.sync_copy(rows_v, out_hbm.at[pl.ds(base + j * CB, CB)])
            return 0

        lax.fori_loop(0, NCH, body, 0)

    return k(node_pad, src_r)


def _sc_scatter(msg, dst):
    dst_r = dst.reshape(NW, NCH, CB)
    zeros = jnp.zeros((N_PAD, 128), dtype=jnp.float32)

    @functools.partial(
        pl.kernel,
        mesh=plsc.VectorSubcoreMesh(**_SC_MESH),
        out_type=jax.ShapeDtypeStruct((NC, N_PAD, 128), jnp.float32),
        scratch_types=[
            pltpu.VMEM((NCH, CB), jnp.int32),
            pltpu.VMEM((CB, 128), jnp.float32),
            pltpu.VMEM_SHARED((N_PAD, 128), jnp.float32),
        ],
    )
    def k(msg_hbm, dst_hbm, z_hbm, out_hbm, idx_v, msg_v, acc_sh):
        cid = lax.axis_index("c")
        sid = lax.axis_index("s")
        wid = sid * NC + cid
        base = wid * EPW
        pltpu.sync_copy(z_hbm.at[pl.ds(sid * NPS, NPS)],
                        acc_sh.at[pl.ds(sid * NPS, NPS)])
        plsc.subcore_barrier()
        pltpu.sync_copy(dst_hbm.at[wid], idx_v)

        def body(j, _):
            pltpu.sync_copy(msg_hbm.at[pl.ds(base + j * CB, CB)], msg_v)
            pltpu.sync_copy(msg_v, acc_sh.at[idx_v.at[j]], add=True)
            return 0

        lax.fori_loop(0, NCH, body, 0)
        plsc.subcore_barrier()
        pltpu.sync_copy(acc_sh.at[pl.ds(sid * NPS, NPS)],
                        out_hbm.at[cid, pl.ds(sid * NPS, NPS)])

    return k(msg, dst_r, zeros)


def _msg_body(ef_ref, xs_ref, w1_ref, b1_ref, w2_ref, b2_ref, erep_ref,
              esum_ref, out_ref):
    h = jnp.maximum(
        jnp.dot(ef_ref[...], w1_ref[...], preferred_element_type=jnp.float32)
        + b1_ref[...], 0.0)
    we = jnp.dot(h.astype(jnp.bfloat16), w2_ref[...],
                 preferred_element_type=jnp.float32) + b2_ref[...]
    xr = jnp.dot(xs_ref[...], erep_ref[...], preferred_element_type=jnp.float32)
    out_ref[...] = jnp.dot(we * xr, esum_ref[...],
                           preferred_element_type=jnp.float32)


def _tc_msg(edge_feats, xs, W1, b1, W2, b2):
    BE = 640
    grid = (E // BE,)
    f = jnp.arange(H)
    erep = (f[None, :] // D_OUT == jnp.arange(128)[:, None]).astype(jnp.float32)
    esum = (f[:, None] % D_OUT == jnp.arange(128)[None, :]).astype(jnp.float32)
    return pl.pallas_call(
        _msg_body,
        grid=grid,
        in_specs=[
            pl.BlockSpec((BE, D_EDGE), lambda i: (i, 0)),
            pl.BlockSpec((BE, 128), lambda i: (i, 0)),
            pl.BlockSpec((D_EDGE, H), lambda i: (0, 0)),
            pl.BlockSpec((1, H), lambda i: (0, 0)),
            pl.BlockSpec((H, H), lambda i: (0, 0)),
            pl.BlockSpec((1, H), lambda i: (0, 0)),
            pl.BlockSpec((128, H), lambda i: (0, 0)),
            pl.BlockSpec((H, 128), lambda i: (0, 0)),
        ],
        out_specs=pl.BlockSpec((BE, 128), lambda i: (i, 0)),
        out_shape=jax.ShapeDtypeStruct((E, 128), jnp.float32),
    )(edge_feats, xs, W1.T, b1[None, :], W2.T.astype(jnp.bfloat16),
      b2[None, :], erep, esum)


def _finish_body(aggp_ref, nf_ref, bconv_ref, wr_ref, wz_ref, wn_ref, br_ref,
                 bz_ref, bni_ref, bnh_ref, wres_ref, bres_ref, gamma_ref,
                 beta_ref, out_ref):
    agg = aggp_ref[0, :N, :D_OUT] + aggp_ref[1, :N, :D_OUT]
    x = jnp.maximum(agg + bconv_ref[...], 0.0)
    r = jax.nn.sigmoid(
        jnp.dot(x, wr_ref[...], preferred_element_type=jnp.float32) + br_ref[...])
    z = jax.nn.sigmoid(
        jnp.dot(x, wz_ref[...], preferred_element_type=jnp.float32) + bz_ref[...])
    n = jnp.tanh(
        jnp.dot(x, wn_ref[...], preferred_element_type=jnp.float32)
        + bni_ref[...] + r * bnh_ref[...])
    hnew = (1.0 - z) * n
    out = hnew + jnp.dot(nf_ref[...], wres_ref[...],
                         preferred_element_type=jnp.float32) + bres_ref[...]
    mean = jnp.mean(out, axis=0, keepdims=True)
    var = jnp.mean((out - mean) ** 2, axis=0, keepdims=True)
    out_ref[...] = (out - mean) * lax.rsqrt(var + 1e-5) * gamma_ref[...] + beta_ref[...]


def _tc_finish(aggp, node_feats, b_conv, Wih, Whh, bih, bhh, Wres, bres,
               gamma, beta):
    O = D_OUT
    args = (
        aggp, node_feats, b_conv[None, :],
        Wih[:O].T, Wih[O:2 * O].T, Wih[2 * O:].T,
        (bih[:O] + bhh[:O])[None, :],
        (bih[O:2 * O] + bhh[O:2 * O])[None, :],
        bih[2 * O:][None, :], bhh[2 * O:][None, :],
        Wres.T, bres[None, :], gamma[None, :], beta[None, :],
    )
    return pl.pallas_call(
        _finish_body,
        out_shape=jax.ShapeDtypeStruct((N, D_OUT), jnp.float32),
    )(*args)


def kernel(node_feats, edge_feats, edge_index, W1, b1, W2, b2, b_conv, Wih,
           Whh, bih, bhh, Wres, bres, gamma, beta):
    src = edge_index[0]
    dst = edge_index[1]
    xs = _sc_gather(node_feats, src)
    msg = _tc_msg(edge_feats, xs, W1, b1, W2, b2)
    aggp = _sc_scatter(msg, dst)
    return _tc_finish(aggp, node_feats, b_conv, Wih, Whh, bih, bhh, Wres,
                      bres, gamma, beta)


# speedup vs baseline: 1.0334x; 1.0334x over previous
import functools

import jax
import jax.numpy as jnp
from jax import lax
from jax.experimental import pallas as pl
from jax.experimental.pallas import tpu as pltpu
from jax.experimental.pallas import tpu_sc as plsc

N = 10000
E = 160000
D_NODE = 32
D_EDGE = 16
D_OUT = 32
H = D_OUT * D_NODE

NC = 2
NS = 16
NW = NC * NS
EPW = E // NW
CB = 40
NCH = EPW // CB
N_PAD = 10240
NPS = N_PAD // NS

_SC_MESH = dict(core_axis_name="c", subcore_axis_name="s")


def _sc_gather(node_feats, src):
    node_pad = jnp.pad(node_feats, ((0, 0), (0, 128 - D_NODE)))
    src_r = src.reshape(NW, NCH, CB)

    @functools.partial(
        pl.kernel,
        mesh=plsc.VectorSubcoreMesh(**_SC_MESH),
        out_type=jax.ShapeDtypeStruct((E, 128), jnp.float32),
        scratch_types=[
            pltpu.VMEM((NCH, CB), jnp.int32),
            pltpu.VMEM((CB, 128), jnp.float32),
            pltpu.SemaphoreType.DMA,
        ],
    )
    def k(node_hbm, src_hbm, out_hbm, idx_v, rows_v, sem):
        cid = lax.axis_index("c")
        sid = lax.axis_index("s")
        wid = sid * NC + cid
        base = wid * EPW
        pltpu.sync_copy(src_hbm.at[wid], idx_v)

        def body(j, _):
            pltpu.async_copy(node_hbm.at[idx_v.at[j]], rows_v, sem).wait()
            pltpu.sync_copy(rows_v, out_hbm.at[pl.ds(base + j * CB, CB)])
            return 0

        lax.fori_loop(0, NCH, body, 0)

    return k(node_pad, src_r)


def _sc_scatter(msg, dst):
    dst_r = dst.reshape(NW, NCH, CB)
    zeros = jnp.zeros((N_PAD, 128), dtype=jnp.float32)

    @functools.partial(
        pl.kernel,
        mesh=plsc.VectorSubcoreMesh(**_SC_MESH),
        out_type=jax.ShapeDtypeStruct((NC, N_PAD, 128), jnp.float32),
        scratch_types=[
            pltpu.VMEM((NCH, CB), jnp.int32),
            pltpu.VMEM((CB, 128), jnp.float32),
            pltpu.VMEM_SHARED((N_PAD, 128), jnp.float32),
        ],
    )
    def k(msg_hbm, dst_hbm, z_hbm, out_hbm, idx_v, msg_v, acc_sh):
        cid = lax.axis_index("c")
        sid = lax.axis_index("s")
        wid = sid * NC + cid
        base = wid * EPW
        pltpu.sync_copy(z_hbm.at[pl.ds(sid * NPS, NPS)],
                        acc_sh.at[pl.ds(sid * NPS, NPS)])
        plsc.subcore_barrier()
        pltpu.sync_copy(dst_hbm.at[wid], idx_v)

        def body(j, _):
            pltpu.sync_copy(msg_hbm.at[pl.ds(base + j * CB, CB)], msg_v)
            pltpu.sync_copy(msg_v, acc_sh.at[idx_v.at[j]], add=True)
            return 0

        lax.fori_loop(0, NCH, body, 0)
        plsc.subcore_barrier()
        pltpu.sync_copy(acc_sh.at[pl.ds(sid * NPS, NPS)],
                        out_hbm.at[cid, pl.ds(sid * NPS, NPS)])

    return k(msg, dst_r, zeros)


def _msg_body(ef_ref, xs_ref, w1_ref, b1_ref, w2_ref, b2_ref, erep_ref,
              esum_ref, out_ref):
    h = jnp.maximum(
        jnp.dot(ef_ref[...], w1_ref[...], preferred_element_type=jnp.float32)
        + b1_ref[...], 0.0)
    we = jnp.dot(h.astype(jnp.bfloat16), w2_ref[...],
                 preferred_element_type=jnp.float32) + b2_ref[...]
    xr = jnp.dot(xs_ref[...].astype(jnp.bfloat16), erep_ref[...],
                 preferred_element_type=jnp.float32)
    out_ref[...] = jnp.dot((we * xr).astype(jnp.bfloat16),
                           esum_ref[...], preferred_element_type=jnp.float32)


def _tc_msg(edge_feats, xs, W1, b1, W2, b2):
    BE = 1280
    grid = (E // BE,)
    f = jnp.arange(H)
    erep = (f[None, :] // D_OUT == jnp.arange(128)[:, None]).astype(jnp.float32)
    esum = (f[:, None] % D_OUT == jnp.arange(128)[None, :]).astype(jnp.float32)
    return pl.pallas_call(
        _msg_body,
        grid=grid,
        in_specs=[
            pl.BlockSpec((BE, D_EDGE), lambda i: (i, 0)),
            pl.BlockSpec((BE, 128), lambda i: (i, 0)),
            pl.BlockSpec((D_EDGE, H), lambda i: (0, 0)),
            pl.BlockSpec((1, H), lambda i: (0, 0)),
            pl.BlockSpec((H, H), lambda i: (0, 0)),
            pl.BlockSpec((1, H), lambda i: (0, 0)),
            pl.BlockSpec((128, H), lambda i: (0, 0)),
            pl.BlockSpec((H, 128), lambda i: (0, 0)),
        ],
        out_specs=pl.BlockSpec((BE, 128), lambda i: (i, 0)),
        out_shape=jax.ShapeDtypeStruct((E, 128), jnp.float32),
    )(edge_feats, xs, W1.T, b1[None, :], W2.T.astype(jnp.bfloat16),
      b2[None, :], erep.astype(jnp.bfloat16), esum.astype(jnp.bfloat16))


def _finish_body(aggp_ref, nf_ref, bconv_ref, wr_ref, wz_ref, wn_ref, br_ref,
                 bz_ref, bni_ref, bnh_ref, wres_ref, bres_ref, gamma_ref,
                 beta_ref, out_ref):
    agg = aggp_ref[0, :N, :D_OUT] + aggp_ref[1, :N, :D_OUT]
    x = jnp.maximum(agg + bconv_ref[...], 0.0)
    r = jax.nn.sigmoid(
        jnp.dot(x, wr_ref[...], preferred_element_type=jnp.float32) + br_ref[...])
    z = jax.nn.sigmoid(
        jnp.dot(x, wz_ref[...], preferred_element_type=jnp.float32) + bz_ref[...])
    n = jnp.tanh(
        jnp.dot(x, wn_ref[...], preferred_element_type=jnp.float32)
        + bni_ref[...] + r * bnh_ref[...])
    hnew = (1.0 - z) * n
    out = hnew + jnp.dot(nf_ref[...], wres_ref[...],
                         preferred_element_type=jnp.float32) + bres_ref[...]
    mean = jnp.mean(out, axis=0, keepdims=True)
    var = jnp.mean((out - mean) ** 2, axis=0, keepdims=True)
    out_ref[...] = (out - mean) * lax.rsqrt(var + 1e-5) * gamma_ref[...] + beta_ref[...]


def _tc_finish(aggp, node_feats, b_conv, Wih, Whh, bih, bhh, Wres, bres,
               gamma, beta):
    O = D_OUT
    args = (
        aggp, node_feats, b_conv[None, :],
        Wih[:O].T, Wih[O:2 * O].T, Wih[2 * O:].T,
        (bih[:O] + bhh[:O])[None, :],
        (bih[O:2 * O] + bhh[O:2 * O])[None, :],
        bih[2 * O:][None, :], bhh[2 * O:][None, :],
        Wres.T, bres[None, :], gamma[None, :], beta[None, :],
    )
    return pl.pallas_call(
        _finish_body,
        out_shape=jax.ShapeDtypeStruct((N, D_OUT), jnp.float32),
    )(*args)


def kernel(node_feats, edge_feats, edge_index, W1, b1, W2, b2, b_conv, Wih,
           Whh, bih, bhh, Wres, bres, gamma, beta):
    src = edge_index[0]
    dst = edge_index[1]
    xs = _sc_gather(node_feats, src)
    msg = _tc_msg(edge_feats, xs, W1, b1, W2, b2)
    aggp = _sc_scatter(msg, dst)
    return _tc_finish(aggp, node_feats, b_conv, Wih, Whh, bih, bhh, Wres,
                      bres, gamma, beta)


# speedup vs baseline: 1.1861x; 1.1477x over previous
import functools

import jax
import jax.numpy as jnp
from jax import lax
from jax.experimental import pallas as pl
from jax.experimental.pallas import tpu as pltpu
from jax.experimental.pallas import tpu_sc as plsc

N = 10000
E = 160000
D_NODE = 32
D_EDGE = 16
D_OUT = 32
H = D_OUT * D_NODE

NC = 2
NS = 16
NW = NC * NS
CB = 40
BE = 1280
N_PAD = 10240
NPS = N_PAD // NS

E_SPLIT = (81920, 78080)

_SC_MESH = dict(core_axis_name="c", subcore_axis_name="s")


def _sc_gather(node_pad, src_h, cnt):
    nch = cnt // (NW * CB)
    epw = nch * CB
    src_r = src_h.reshape(NW, nch, CB)

    @functools.partial(
        pl.kernel,
        mesh=plsc.VectorSubcoreMesh(**_SC_MESH),
        out_type=jax.ShapeDtypeStruct((cnt, 128), jnp.float32),
        scratch_types=[
            pltpu.VMEM((nch, CB), jnp.int32),
            pltpu.VMEM((CB, 128), jnp.float32),
            pltpu.SemaphoreType.DMA,
        ],
    )
    def k(node_hbm, src_hbm, out_hbm, idx_v, rows_v, sem):
        cid = lax.axis_index("c")
        sid = lax.axis_index("s")
        wid = sid * NC + cid
        base = wid * epw
        pltpu.sync_copy(src_hbm.at[wid], idx_v)

        def body(j, _):
            pltpu.async_copy(node_hbm.at[idx_v.at[j]], rows_v, sem).wait()
            pltpu.sync_copy(rows_v, out_hbm.at[pl.ds(base + j * CB, CB)])
            return 0

        lax.fori_loop(0, nch, body, 0)

    return k(node_pad, src_r)


def _sc_scatter(msg, dst_h, cnt):
    nch = cnt // (NW * CB)
    epw = nch * CB
    dst_r = dst_h.reshape(NW, nch, CB)
    zeros = jnp.zeros((N_PAD, 128), dtype=jnp.float32)

    @functools.partial(
        pl.kernel,
        mesh=plsc.VectorSubcoreMesh(**_SC_MESH),
        out_type=jax.ShapeDtypeStruct((NC, N_PAD, 128), jnp.float32),
        scratch_types=[
            pltpu.VMEM((nch, CB), jnp.int32),
            pltpu.VMEM((CB, 128), jnp.float32),
            pltpu.VMEM_SHARED((N_PAD, 128), jnp.float32),
        ],
    )
    def k(msg_hbm, dst_hbm, z_hbm, out_hbm, idx_v, msg_v, acc_sh):
        cid = lax.axis_index("c")
        sid = lax.axis_index("s")
        wid = sid * NC + cid
        base = wid * epw
        pltpu.sync_copy(z_hbm.at[pl.ds(sid * NPS, NPS)],
                        acc_sh.at[pl.ds(sid * NPS, NPS)])
        plsc.subcore_barrier()
        pltpu.sync_copy(dst_hbm.at[wid], idx_v)

        def body(j, _):
            pltpu.sync_copy(msg_hbm.at[pl.ds(base + j * CB, CB)], msg_v)
            pltpu.sync_copy(msg_v, acc_sh.at[idx_v.at[j]], add=True)
            return 0

        lax.fori_loop(0, nch, body, 0)
        plsc.subcore_barrier()
        pltpu.sync_copy(acc_sh.at[pl.ds(sid * NPS, NPS)],
                        out_hbm.at[cid, pl.ds(sid * NPS, NPS)])

    return k(msg, dst_r, zeros)


def _msg_body(ef_ref, xs_ref, w1_ref, b1_ref, w2_ref, b2_ref, erep_ref,
              esum_ref, out_ref):
    h = jnp.maximum(
        jnp.dot(ef_ref[...], w1_ref[...], preferred_element_type=jnp.float32)
        + b1_ref[...], 0.0)
    we = jnp.dot(h.astype(jnp.bfloat16), w2_ref[...],
                 preferred_element_type=jnp.float32).astype(jnp.bfloat16) + b2_ref[...]
    xr = jnp.dot(xs_ref[...].astype(jnp.bfloat16), erep_ref[...],
                 preferred_element_type=jnp.float32).astype(jnp.bfloat16)
    out_ref[...] = jnp.dot(we * xr, esum_ref[...],
                           preferred_element_type=jnp.float32)


def _tc_msg(edge_feats, xs_h, e0, cnt, W1, b1, W2, b2):
    grid = (cnt // BE,)
    i0 = e0 // BE
    f = jnp.arange(H)
    erep = (f[None, :] // D_OUT == jnp.arange(128)[:, None]).astype(jnp.bfloat16)
    esum = (f[:, None] % D_OUT == jnp.arange(128)[None, :]).astype(jnp.bfloat16)
    return pl.pallas_call(
        _msg_body,
        grid=grid,
        in_specs=[
            pl.BlockSpec((BE, D_EDGE), lambda i: (i + i0, 0)),
            pl.BlockSpec((BE, 128), lambda i: (i, 0)),
            pl.BlockSpec((D_EDGE, H), lambda i: (0, 0)),
            pl.BlockSpec((1, H), lambda i: (0, 0)),
            pl.BlockSpec((H, H), lambda i: (0, 0)),
            pl.BlockSpec((1, H), lambda i: (0, 0)),
            pl.BlockSpec((128, H), lambda i: (0, 0)),
            pl.BlockSpec((H, 128), lambda i: (0, 0)),
        ],
        out_specs=pl.BlockSpec((BE, 128), lambda i: (i, 0)),
        out_shape=jax.ShapeDtypeStruct((cnt, 128), jnp.float32),
    )(edge_feats, xs_h, W1.T, b1[None, :], W2.T.astype(jnp.bfloat16),
      b2[None, :].astype(jnp.bfloat16), erep, esum)


def _finish_body(agg1_ref, agg2_ref, nf_ref, bconv_ref, wr_ref, wz_ref,
                 wn_ref, br_ref, bz_ref, bni_ref, bnh_ref, wres_ref, bres_ref,
                 gamma_ref, beta_ref, out_ref):
    agg = (agg1_ref[0, :N, :D_OUT] + agg1_ref[1, :N, :D_OUT]
           + agg2_ref[0, :N, :D_OUT] + agg2_ref[1, :N, :D_OUT])
    x = jnp.maximum(agg + bconv_ref[...], 0.0)
    r = jax.nn.sigmoid(
        jnp.dot(x, wr_ref[...], preferred_element_type=jnp.float32) + br_ref[...])
    z = jax.nn.sigmoid(
        jnp.dot(x, wz_ref[...], preferred_element_type=jnp.float32) + bz_ref[...])
    n = jnp.tanh(
        jnp.dot(x, wn_ref[...], preferred_element_type=jnp.float32)
        + bni_ref[...] + r * bnh_ref[...])
    hnew = (1.0 - z) * n
    out = hnew + jnp.dot(nf_ref[...], wres_ref[...],
                         preferred_element_type=jnp.float32) + bres_ref[...]
    mean = jnp.mean(out, axis=0, keepdims=True)
    var = jnp.mean((out - mean) ** 2, axis=0, keepdims=True)
    out_ref[...] = (out - mean) * lax.rsqrt(var + 1e-5) * gamma_ref[...] + beta_ref[...]


def _tc_finish(aggp1, aggp2, node_feats, b_conv, Wih, Whh, bih, bhh, Wres,
               bres, gamma, beta):
    O = D_OUT
    args = (
        aggp1, aggp2, node_feats, b_conv[None, :],
        Wih[:O].T, Wih[O:2 * O].T, Wih[2 * O:].T,
        (bih[:O] + bhh[:O])[None, :],
        (bih[O:2 * O] + bhh[O:2 * O])[None, :],
        bih[2 * O:][None, :], bhh[2 * O:][None, :],
        Wres.T, bres[None, :], gamma[None, :], beta[None, :],
    )
    return pl.pallas_call(
        _finish_body,
        out_shape=jax.ShapeDtypeStruct((N, D_OUT), jnp.float32),
    )(*args)


def kernel(node_feats, edge_feats, edge_index, W1, b1, W2, b2, b_conv, Wih,
           Whh, bih, bhh, Wres, bres, gamma, beta):
    src = edge_index[0]
    dst = edge_index[1]
    node_pad = jnp.pad(node_feats, ((0, 0), (0, 128 - D_NODE)))

    e1, e2 = E_SPLIT
    xs1 = _sc_gather(node_pad, src[:e1], e1)
    xs2 = _sc_gather(node_pad, src[e1:], e2)
    msg1 = _tc_msg(edge_feats, xs1, 0, e1, W1, b1, W2, b2)
    msg2 = _tc_msg(edge_feats, xs2, e1, e2, W1, b1, W2, b2)
    aggp1 = _sc_scatter(msg1, dst[:e1], e1)
    aggp2 = _sc_scatter(msg2, dst[e1:], e2)
    return _tc_finish(aggp1, aggp2, node_feats, b_conv, Wih, Whh, bih, bhh,
                      Wres, bres, gamma, beta)


# speedup vs baseline: 1.2545x; 1.0577x over previous
import functools

import jax
import jax.numpy as jnp
from jax import lax
from jax.experimental import pallas as pl
from jax.experimental.pallas import tpu as pltpu
from jax.experimental.pallas import tpu_sc as plsc

N = 10000
E = 160000
D_NODE = 32
D_EDGE = 16
D_OUT = 32
H = D_OUT * D_NODE

NC = 2
NS = 16
NW = NC * NS
CB = 40
BE = 1280
N_PAD = 10240
NPS = N_PAD // NS

E_SPLIT = (40960, 39680, 39680, 39680)

_SC_MESH = dict(core_axis_name="c", subcore_axis_name="s")


def _sc_gather(node_pad, src_h, cnt):
    nch = cnt // (NW * CB)
    epw = nch * CB
    src_r = src_h.reshape(NW, nch, CB)

    @functools.partial(
        pl.kernel,
        mesh=plsc.VectorSubcoreMesh(**_SC_MESH),
        out_type=jax.ShapeDtypeStruct((cnt, 128), jnp.float32),
        scratch_types=[
            pltpu.VMEM((nch, CB), jnp.int32),
            pltpu.VMEM((CB, 128), jnp.float32),
            pltpu.SemaphoreType.DMA,
        ],
    )
    def k(node_hbm, src_hbm, out_hbm, idx_v, rows_v, sem):
        cid = lax.axis_index("c")
        sid = lax.axis_index("s")
        wid = sid * NC + cid
        base = wid * epw
        pltpu.sync_copy(src_hbm.at[wid], idx_v)

        def body(j, _):
            pltpu.async_copy(node_hbm.at[idx_v.at[j]], rows_v, sem).wait()
            pltpu.sync_copy(rows_v, out_hbm.at[pl.ds(base + j * CB, CB)])
            return 0

        lax.fori_loop(0, nch, body, 0)

    return k(node_pad, src_r)


def _sc_scatter(msg, dst_h, cnt, prev):
    nch = cnt // (NW * CB)
    epw = nch * CB
    dst_r = dst_h.reshape(NW, nch, CB)

    @functools.partial(
        pl.kernel,
        mesh=plsc.VectorSubcoreMesh(**_SC_MESH),
        out_type=jax.ShapeDtypeStruct((NC, N_PAD, 128), jnp.float32),
        scratch_types=[
            pltpu.VMEM((nch, CB), jnp.int32),
            pltpu.VMEM((CB, 128), jnp.float32),
            pltpu.VMEM_SHARED((N_PAD, 128), jnp.float32),
        ],
    )
    def k(msg_hbm, dst_hbm, prev_hbm, out_hbm, idx_v, msg_v, acc_sh):
        cid = lax.axis_index("c")
        sid = lax.axis_index("s")
        wid = sid * NC + cid
        base = wid * epw
        pltpu.sync_copy(prev_hbm.at[cid, pl.ds(sid * NPS, NPS)],
                        acc_sh.at[pl.ds(sid * NPS, NPS)])
        plsc.subcore_barrier()
        pltpu.sync_copy(dst_hbm.at[wid], idx_v)

        def body(j, _):
            pltpu.sync_copy(msg_hbm.at[pl.ds(base + j * CB, CB)], msg_v)
            pltpu.sync_copy(msg_v, acc_sh.at[idx_v.at[j]], add=True)
            return 0

        lax.fori_loop(0, nch, body, 0)
        plsc.subcore_barrier()
        pltpu.sync_copy(acc_sh.at[pl.ds(sid * NPS, NPS)],
                        out_hbm.at[cid, pl.ds(sid * NPS, NPS)])

    return k(msg, dst_r, prev)


def _msg_body(ef_ref, xs_ref, w1_ref, b1_ref, w2_ref, b2_ref, erep_ref,
              esum_ref, out_ref):
    h = jnp.maximum(
        jnp.dot(ef_ref[...], w1_ref[...], preferred_element_type=jnp.float32)
        + b1_ref[...], 0.0)
    we = jnp.dot(h.astype(jnp.bfloat16), w2_ref[...],
                 preferred_element_type=jnp.float32).astype(jnp.bfloat16) + b2_ref[...]
    xr = jnp.dot(xs_ref[...].astype(jnp.bfloat16), erep_ref[...],
                 preferred_element_type=jnp.float32).astype(jnp.bfloat16)
    out_ref[...] = jnp.dot(we * xr, esum_ref[...],
                           preferred_element_type=jnp.float32)


def _tc_msg(edge_feats, xs_h, e0, cnt, W1, b1, W2, b2):
    grid = (cnt // BE,)
    i0 = e0 // BE
    f = jnp.arange(H)
    erep = (f[None, :] // D_OUT == jnp.arange(128)[:, None]).astype(jnp.bfloat16)
    esum = (f[:, None] % D_OUT == jnp.arange(128)[None, :]).astype(jnp.bfloat16)
    return pl.pallas_call(
        _msg_body,
        grid=grid,
        in_specs=[
            pl.BlockSpec((BE, D_EDGE), lambda i: (i + i0, 0)),
            pl.BlockSpec((BE, 128), lambda i: (i, 0)),
            pl.BlockSpec((D_EDGE, H), lambda i: (0, 0)),
            pl.BlockSpec((1, H), lambda i: (0, 0)),
            pl.BlockSpec((H, H), lambda i: (0, 0)),
            pl.BlockSpec((1, H), lambda i: (0, 0)),
            pl.BlockSpec((128, H), lambda i: (0, 0)),
            pl.BlockSpec((H, 128), lambda i: (0, 0)),
        ],
        out_specs=pl.BlockSpec((BE, 128), lambda i: (i, 0)),
        out_shape=jax.ShapeDtypeStruct((cnt, 128), jnp.float32),
    )(edge_feats, xs_h, W1.T, b1[None, :], W2.T.astype(jnp.bfloat16),
      b2[None, :].astype(jnp.bfloat16), erep, esum)


def _finish_body(aggp_ref, nf_ref, bconv_ref, wr_ref, wz_ref, wn_ref,
                 br_ref, bz_ref, bni_ref, bnh_ref, wres_ref, bres_ref,
                 gamma_ref, beta_ref, out_ref):
    agg = aggp_ref[0, :N, :D_OUT] + aggp_ref[1, :N, :D_OUT]
    x = jnp.maximum(agg + bconv_ref[...], 0.0)
    r = jax.nn.sigmoid(
        jnp.dot(x, wr_ref[...], preferred_element_type=jnp.float32) + br_ref[...])
    z = jax.nn.sigmoid(
        jnp.dot(x, wz_ref[...], preferred_element_type=jnp.float32) + bz_ref[...])
    n = jnp.tanh(
        jnp.dot(x, wn_ref[...], preferred_element_type=jnp.float32)
        + bni_ref[...] + r * bnh_ref[...])
    hnew = (1.0 - z) * n
    out = hnew + jnp.dot(nf_ref[...], wres_ref[...],
                         preferred_element_type=jnp.float32) + bres_ref[...]
    mean = jnp.mean(out, axis=0, keepdims=True)
    var = jnp.mean((out - mean) ** 2, axis=0, keepdims=True)
    out_ref[...] = (out - mean) * lax.rsqrt(var + 1e-5) * gamma_ref[...] + beta_ref[...]


def _tc_finish(aggp, node_feats, b_conv, Wih, Whh, bih, bhh, Wres,
               bres, gamma, beta):
    O = D_OUT
    args = (
        aggp, node_feats, b_conv[None, :],
        Wih[:O].T, Wih[O:2 * O].T, Wih[2 * O:].T,
        (bih[:O] + bhh[:O])[None, :],
        (bih[O:2 * O] + bhh[O:2 * O])[None, :],
        bih[2 * O:][None, :], bhh[2 * O:][None, :],
        Wres.T, bres[None, :], gamma[None, :], beta[None, :],
    )
    return pl.pallas_call(
        _finish_body,
        out_shape=jax.ShapeDtypeStruct((N, D_OUT), jnp.float32),
    )(*args)


def kernel(node_feats, edge_feats, edge_index, W1, b1, W2, b2, b_conv, Wih,
           Whh, bih, bhh, Wres, bres, gamma, beta):
    src = edge_index[0]
    dst = edge_index[1]
    node_pad = jnp.pad(node_feats, ((0, 0), (0, 128 - D_NODE)))

    offs = [0]
    for c in E_SPLIT:
        offs.append(offs[-1] + c)
    xss, msgs = [], []
    for i, cnt in enumerate(E_SPLIT):
        xss.append(_sc_gather(node_pad, src[offs[i]:offs[i + 1]], cnt))
    for i, cnt in enumerate(E_SPLIT):
        msgs.append(_tc_msg(edge_feats, xss[i], offs[i], cnt, W1, b1, W2, b2))
    aggp = jnp.zeros((NC, N_PAD, 128), dtype=jnp.float32)
    for i, cnt in enumerate(E_SPLIT):
        aggp = _sc_scatter(msgs[i], dst[offs[i]:offs[i + 1]], cnt, aggp)
    return _tc_finish(aggp, node_feats, b_conv, Wih, Whh, bih, bhh,
                      Wres, bres, gamma, beta)


# speedup vs baseline: 1.2850x; 1.0243x over previous
import functools

import jax
import jax.numpy as jnp
from jax import lax
from jax.experimental import pallas as pl
from jax.experimental.pallas import tpu as pltpu
from jax.experimental.pallas import tpu_sc as plsc

N = 10000
E = 160000
D_NODE = 32
D_EDGE = 16
D_OUT = 32
H = D_OUT * D_NODE

NC = 2
NS = 16
NW = NC * NS
CB = 40
BE = 1280
N_PAD = 10240
NPS = N_PAD // NS

E_SPLIT = ((37120, 40), (40960, 128), (40960, 128), (40960, 128))

_SC_MESH = dict(core_axis_name="c", subcore_axis_name="s")


def _sc_gather(node_pad, src_h, cnt, cb):
    nch = cnt // (NW * cb)
    epw = nch * cb
    src_r = src_h.reshape(NW, nch, cb)

    @functools.partial(
        pl.kernel,
        mesh=plsc.VectorSubcoreMesh(**_SC_MESH),
        out_type=jax.ShapeDtypeStruct((cnt, 128), jnp.float32),
        scratch_types=[
            pltpu.VMEM((nch, cb), jnp.int32),
            pltpu.VMEM((2, cb, 128), jnp.float32),
            pltpu.SemaphoreType.DMA,
            pltpu.SemaphoreType.DMA,
        ],
    )
    def k(node_hbm, src_hbm, out_hbm, idx_v, rows_v, sg0, sg1):
        cid = lax.axis_index("c")
        sid = lax.axis_index("s")
        wid = sid * NC + cid
        base = wid * epw
        pltpu.sync_copy(src_hbm.at[wid], idx_v)
        pltpu.async_copy(node_hbm.at[idx_v.at[0]], rows_v.at[0], sg0)

        def pair(p, _):
            j0 = 2 * p
            j1 = j0 + 1
            pltpu.async_copy(node_hbm.at[idx_v.at[j1]], rows_v.at[1], sg1)
            pltpu.make_async_copy(node_hbm.at[idx_v.at[j0]],
                                  rows_v.at[0], sg0).wait()
            pltpu.sync_copy(rows_v.at[0], out_hbm.at[pl.ds(base + j0 * cb, cb)])

            @pl.when(j1 + 1 < nch)
            def _():
                pltpu.async_copy(node_hbm.at[idx_v.at[j1 + 1]], rows_v.at[0],
                                 sg0)

            pltpu.make_async_copy(node_hbm.at[idx_v.at[j1]],
                                  rows_v.at[1], sg1).wait()
            pltpu.sync_copy(rows_v.at[1], out_hbm.at[pl.ds(base + j1 * cb, cb)])
            return 0

        lax.fori_loop(0, nch // 2, pair, 0)
        if nch % 2:
            j = nch - 1
            pltpu.make_async_copy(node_hbm.at[idx_v.at[j]],
                                  rows_v.at[0], sg0).wait()
            pltpu.sync_copy(rows_v.at[0], out_hbm.at[pl.ds(base + j * cb, cb)])

    return k(node_pad, src_r)


def _sc_scatter(msg, dst_h, cnt, cb, prev):
    nch = cnt // (NW * cb)
    epw = nch * cb
    dst_r = dst_h.reshape(NW, nch, cb)

    @functools.partial(
        pl.kernel,
        mesh=plsc.VectorSubcoreMesh(**_SC_MESH),
        out_type=jax.ShapeDtypeStruct((NC, N_PAD, 128), jnp.float32),
        scratch_types=[
            pltpu.VMEM((nch, cb), jnp.int32),
            pltpu.VMEM((2, cb, 128), jnp.float32),
            pltpu.VMEM_SHARED((N_PAD, 128), jnp.float32),
            pltpu.SemaphoreType.DMA,
            pltpu.SemaphoreType.DMA,
        ],
    )
    def k(msg_hbm, dst_hbm, prev_hbm, out_hbm, idx_v, msg_v, acc_sh, sl0, sl1):
        cid = lax.axis_index("c")
        sid = lax.axis_index("s")
        wid = sid * NC + cid
        base = wid * epw
        pltpu.sync_copy(prev_hbm.at[cid, pl.ds(sid * NPS, NPS)],
                        acc_sh.at[pl.ds(sid * NPS, NPS)])
        plsc.subcore_barrier()
        pltpu.sync_copy(dst_hbm.at[wid], idx_v)
        pltpu.async_copy(msg_hbm.at[pl.ds(base, cb)], msg_v.at[0], sl0)

        def pair(p, _):
            j0 = 2 * p
            j1 = j0 + 1
            pltpu.async_copy(msg_hbm.at[pl.ds(base + j1 * cb, cb)],
                             msg_v.at[1], sl1)
            pltpu.make_async_copy(msg_hbm.at[pl.ds(base + j0 * cb, cb)],
                                  msg_v.at[0], sl0).wait()
            pltpu.sync_copy(msg_v.at[0], acc_sh.at[idx_v.at[j0]], add=True)

            @pl.when(j1 + 1 < nch)
            def _():
                pltpu.async_copy(msg_hbm.at[pl.ds(base + (j1 + 1) * cb, cb)],
                                 msg_v.at[0], sl0)

            pltpu.make_async_copy(msg_hbm.at[pl.ds(base + j1 * cb, cb)],
                                  msg_v.at[1], sl1).wait()
            pltpu.sync_copy(msg_v.at[1], acc_sh.at[idx_v.at[j1]], add=True)
            return 0

        lax.fori_loop(0, nch // 2, pair, 0)
        if nch % 2:
            j = nch - 1
            pltpu.make_async_copy(msg_hbm.at[pl.ds(base + j * cb, cb)],
                                  msg_v.at[0], sl0).wait()
            pltpu.sync_copy(msg_v.at[0], acc_sh.at[idx_v.at[j]], add=True)
        plsc.subcore_barrier()
        pltpu.sync_copy(acc_sh.at[pl.ds(sid * NPS, NPS)],
                        out_hbm.at[cid, pl.ds(sid * NPS, NPS)])

    return k(msg, dst_r, prev)


def _msg_body(ef_ref, xs_ref, w1_ref, b1_ref, w2_ref, b2_ref, erep_ref,
              esum_ref, out_ref):
    h = jnp.maximum(
        jnp.dot(ef_ref[...], w1_ref[...], preferred_element_type=jnp.float32)
        + b1_ref[...], 0.0)
    we = jnp.dot(h.astype(jnp.bfloat16), w2_ref[...],
                 preferred_element_type=jnp.float32).astype(jnp.bfloat16) + b2_ref[...]
    xr = jnp.dot(xs_ref[...].astype(jnp.bfloat16), erep_ref[...],
                 preferred_element_type=jnp.float32).astype(jnp.bfloat16)
    out_ref[...] = jnp.dot(we * xr, esum_ref[...],
                           preferred_element_type=jnp.float32)


def _tc_msg(edge_feats, xs_h, e0, cnt, W1, b1, W2, b2):
    grid = (cnt // BE,)
    i0 = e0 // BE
    f = jnp.arange(H)
    erep = (f[None, :] // D_OUT == jnp.arange(128)[:, None]).astype(jnp.bfloat16)
    esum = (f[:, None] % D_OUT == jnp.arange(128)[None, :]).astype(jnp.bfloat16)
    return pl.pallas_call(
        _msg_body,
        grid=grid,
        in_specs=[
            pl.BlockSpec((BE, D_EDGE), lambda i: (i + i0, 0)),
            pl.BlockSpec((BE, 128), lambda i: (i, 0)),
            pl.BlockSpec((D_EDGE, H), lambda i: (0, 0)),
            pl.BlockSpec((1, H), lambda i: (0, 0)),
            pl.BlockSpec((H, H), lambda i: (0, 0)),
            pl.BlockSpec((1, H), lambda i: (0, 0)),
            pl.BlockSpec((128, H), lambda i: (0, 0)),
            pl.BlockSpec((H, 128), lambda i: (0, 0)),
        ],
        out_specs=pl.BlockSpec((BE, 128), lambda i: (i, 0)),
        out_shape=jax.ShapeDtypeStruct((cnt, 128), jnp.float32),
    )(edge_feats, xs_h, W1.T, b1[None, :], W2.T.astype(jnp.bfloat16),
      b2[None, :].astype(jnp.bfloat16), erep, esum)


def _finish_body(aggp_ref, nf_ref, bconv_ref, wr_ref, wz_ref, wn_ref,
                 br_ref, bz_ref, bni_ref, bnh_ref, wres_ref, bres_ref,
                 gamma_ref, beta_ref, out_ref):
    agg = aggp_ref[0, :N, :D_OUT] + aggp_ref[1, :N, :D_OUT]
    x = jnp.maximum(agg + bconv_ref[...], 0.0)
    r = jax.nn.sigmoid(
        jnp.dot(x, wr_ref[...], preferred_element_type=jnp.float32) + br_ref[...])
    z = jax.nn.sigmoid(
        jnp.dot(x, wz_ref[...], preferred_element_type=jnp.float32) + bz_ref[...])
    n = jnp.tanh(
        jnp.dot(x, wn_ref[...], preferred_element_type=jnp.float32)
        + bni_ref[...] + r * bnh_ref[...])
    hnew = (1.0 - z) * n
    out = hnew + jnp.dot(nf_ref[...], wres_ref[...],
                         preferred_element_type=jnp.float32) + bres_ref[...]
    mean = jnp.mean(out, axis=0, keepdims=True)
    var = jnp.mean((out - mean) ** 2, axis=0, keepdims=True)
    out_ref[...] = (out - mean) * lax.rsqrt(var + 1e-5) * gamma_ref[...] + beta_ref[...]


def _tc_finish(aggp, node_feats, b_conv, Wih, Whh, bih, bhh, Wres,
               bres, gamma, beta):
    O = D_OUT
    args = (
        aggp, node_feats, b_conv[None, :],
        Wih[:O].T, Wih[O:2 * O].T, Wih[2 * O:].T,
        (bih[:O] + bhh[:O])[None, :],
        (bih[O:2 * O] + bhh[O:2 * O])[None, :],
        bih[2 * O:][None, :], bhh[2 * O:][None, :],
        Wres.T, bres[None, :], gamma[None, :], beta[None, :],
    )
    return pl.pallas_call(
        _finish_body,
        out_shape=jax.ShapeDtypeStruct((N, D_OUT), jnp.float32),
    )(*args)


def kernel(node_feats, edge_feats, edge_index, W1, b1, W2, b2, b_conv, Wih,
           Whh, bih, bhh, Wres, bres, gamma, beta):
    src = edge_index[0]
    dst = edge_index[1]
    node_pad = jnp.pad(node_feats, ((0, 0), (0, 128 - D_NODE)))

    offs = [0]
    for c, _ in E_SPLIT:
        offs.append(offs[-1] + c)
    xss, msgs = [], []
    for i, (cnt, cb) in enumerate(E_SPLIT):
        xss.append(_sc_gather(node_pad, src[offs[i]:offs[i + 1]], cnt, cb))
    for i, (cnt, cb) in enumerate(E_SPLIT):
        msgs.append(_tc_msg(edge_feats, xss[i], offs[i], cnt, W1, b1, W2, b2))
    aggp = jnp.zeros((NC, N_PAD, 128), dtype=jnp.float32)
    for i, (cnt, cb) in enumerate(E_SPLIT):
        aggp = _sc_scatter(msgs[i], dst[offs[i]:offs[i + 1]], cnt, cb, aggp)
    return _tc_finish(aggp, node_feats, b_conv, Wih, Whh, bih, bhh,
                      Wres, bres, gamma, beta)
